# baseline (speedup 1.0000x reference)
import functools

import jax
import jax.numpy as jnp
from jax import lax
from jax.experimental import pallas as pl
from jax.experimental.pallas import tpu as pltpu
from jax.experimental.pallas import tpu_sc as plsc

F32 = jnp.float32
E = 128
BLK = 256
NC, NS = 2, 16
NW = NC * NS
CHUNK = 128

_L = (500, 4500, 15000, 80000)
_P = (512, 4608, 15360, 81920)
_O = (0, 512, 5120, 20480)
_T = 102400


def _mesh():
    return plsc.VectorSubcoreMesh(core_axis_name="c", subcore_axis_name="s")


_NB = 4


def _sc_gather(src, idx, n_rows):
    q = n_rows // NW
    nfull, r = divmod(q, CHUNK)
    nq, rem = divmod(nfull, _NB)

    scratch = [pltpu.VMEM((q,), jnp.int32)]
    scratch += [pltpu.VMEM((CHUNK, E), F32)] * _NB
    if r:
        scratch += [pltpu.VMEM((r, E), F32)]
    scratch += [pltpu.SemaphoreType.DMA] * (2 * _NB + 1)

    @functools.partial(
        pl.kernel, mesh=_mesh(),
        out_type=jax.ShapeDtypeStruct((n_rows, E), F32),
        scratch_types=scratch,
    )
    def k(src_hbm, idx_hbm, out_hbm, *sc):
        ia = sc[0]
        v = sc[1:1 + _NB]
        vt = sc[1 + _NB] if r else None
        sems = sc[-(2 * _NB + 1):]
        g = sems[:_NB]
        w = sems[_NB:2 * _NB]
        st = sems[2 * _NB]
        base = (lax.axis_index("c") * NS + lax.axis_index("s")) * q
        pltpu.sync_copy(idx_hbm.at[pl.ds(base, q)], ia)

        def start(c, b):
            pltpu.async_copy(
                src_hbm.at[ia.at[pl.ds(c * CHUNK, CHUNK)]], v[b], g[b])

        def wback(c, b):
            pltpu.async_copy(
                v[b], out_hbm.at[pl.ds(base + c * CHUNK, CHUNK)], w[b])

        def wait_g(b):
            pltpu.make_async_copy(src_hbm.at[pl.ds(0, CHUNK)], v[b],
                                  g[b]).wait()

        def wait_w(b):
            pltpu.make_async_copy(v[b], out_hbm.at[pl.ds(base, CHUNK)],
                                  w[b]).wait()

        if nq:
            for b in range(_NB):
                start(b, b)

            @pl.loop(0, nq - 1)
            def _(j):
                c = j * _NB
                for b in range(_NB):
                    wait_g(b)
                    wback(c + b, b)
                for b in range(_NB):
                    wait_w(b)
                    start(c + _NB + b, b)

            for b in range(_NB):
                wait_g(b)
                wback((nq - 1) * _NB + b, b)
            for b in range(rem):
                wait_w(b)
                start(nq * _NB + b, b)
            for b in range(rem):
                wait_g(b)
                wback(nq * _NB + b, b)
            for b in range(rem, _NB):
                wait_w(b)
            for b in range(rem):
                wait_w(b)
        else:
            for b in range(rem):
                start(b, b)
            for b in range(rem):
                wait_g(b)
                wback(b, b)
            for b in range(rem):
                wait_w(b)
        if r:
            t = nfull * CHUNK
            pltpu.async_copy(
                src_hbm.at[ia.at[pl.ds(t, r)]], vt, st).wait()
            pltpu.sync_copy(vt, out_hbm.at[pl.ds(base + t, r)])

    return k(src, idx)


def _scatter_plan(par, n_par, n_child):
    h = n_par // 2
    s = jnp.searchsorted(par, h).astype(jnp.int32)
    nc0 = (s + CHUNK - 1) // CHUNK
    base1 = s // CHUNK
    ntot = n_child // CHUNK
    idx0 = jnp.where(par < h, par, h)
    idx1 = jnp.where(par >= h, par - h, h)
    idx2 = jnp.concatenate([idx0, idx1])
    params = jnp.stack([nc0, base1] + [jnp.int32(0)] * 14)
    return idx2, params


def _sc_scatter_add(vals, idx2, params, n_par):
    n_child = vals.shape[0]
    h = n_par // 2
    z = h // NS
    ntot = n_child // CHUNK
    zeros = jnp.zeros((z, E), F32)

    scratch = [pltpu.VMEM((16,), jnp.int32),
               pltpu.VMEM((CHUNK,), jnp.int32), pltpu.VMEM((CHUNK, E), F32),
               pltpu.VMEM((CHUNK,), jnp.int32), pltpu.VMEM((CHUNK, E), F32),
               pltpu.VMEM_SHARED((h + 8, E), F32),
               pltpu.SemaphoreType.DMA, pltpu.SemaphoreType.DMA]

    @functools.partial(
        pl.kernel, mesh=_mesh(),
        out_type=jax.ShapeDtypeStruct((n_par, E), F32),
        scratch_types=scratch,
    )
    def k(vals_hbm, idx_hbm, params_hbm, zeros_hbm, out_hbm, *sc):
        pv, i0, v0, i1, v1, shared, s0, s1 = sc
        cid = lax.axis_index("c")
        sid = lax.axis_index("s")
        pltpu.sync_copy(params_hbm, pv)
        pltpu.sync_copy(zeros_hbm, shared.at[pl.ds(sid * z, z)])
        plsc.subcore_barrier()

        pvec = pv[...]
        nc0 = pvec[0]
        base1 = pvec[1]
        base_c = jnp.where(cid == 0, 0, base1)
        nc_c = jnp.where(cid == 0, nc0, ntot - base1)
        m = jnp.maximum(nc_c - sid + NS - 1, 0) // NS
        ioff = cid * n_child

        def pair(j, carry):
            g0 = base_c + sid + (2 * j) * NS
            a = g0 * CHUNK
            pltpu.sync_copy(idx_hbm.at[pl.ds(ioff + a, CHUNK)], i0)
            h0 = pltpu.async_copy(vals_hbm.at[pl.ds(a, CHUNK)], v0, s0)
            second = (2 * j + 1) < m

            @pl.when(second)
            def _():
                b = a + NS * CHUNK
                pltpu.sync_copy(idx_hbm.at[pl.ds(ioff + b, CHUNK)], i1)
                h1 = pltpu.async_copy(vals_hbm.at[pl.ds(b, CHUNK)], v1, s1)
                h0.wait()
                pltpu.sync_copy(v0, shared.at[i0], add=True)
                h1.wait()
                pltpu.sync_copy(v1, shared.at[i1], add=True)

            @pl.when(jnp.logical_not(second))
            def _():
                h0.wait()
                pltpu.sync_copy(v0, shared.at[i0], add=True)

            return carry

        lax.fori_loop(0, (m + 1) // 2, pair, 0)

        plsc.subcore_barrier()
        pltpu.sync_copy(shared.at[pl.ds(sid * z, z)],
                        out_hbm.at[pl.ds(cid * h + sid * z, z)])

    return k(vals, idx2, params, zeros)


_NPAR = _O[3]


def _pf_body(x_ref, wf_ref, pf_ref):
    pf_ref[...] = jnp.dot(x_ref[...], wf_ref[...], preferred_element_type=F32)


def _tc_pf(x, w_f):
    return pl.pallas_call(
        _pf_body,
        grid=(_NPAR // BLK,),
        in_specs=[
            pl.BlockSpec((BLK, E), lambda i: (i, 0)),
            pl.BlockSpec((E, E), lambda i: (0, 0)),
        ],
        out_specs=pl.BlockSpec((BLK, E), lambda i: (i, 0)),
        out_shape=jax.ShapeDtypeStruct((_NPAR, E), F32),
    )(x, w_f)


def _fc(h, c, g_ref, uf_ref, bf_ref):
    hu = jnp.dot(h, uf_ref[...], preferred_element_type=F32)
    return jax.nn.sigmoid(g_ref[...] + hu + bf_ref[...]) * c


def _leaf_body(x_ref, g_ref, wiou_ref, uf_ref, biou_ref, bf_ref,
               h_ref, fc_ref):
    iou = jnp.dot(x_ref[...], wiou_ref[...], preferred_element_type=F32)
    iou = iou + biou_ref[...]
    i, o, u = jnp.split(iou, 3, axis=-1)
    c = jax.nn.sigmoid(i) * jnp.tanh(u)
    h = jax.nn.sigmoid(o) * jnp.tanh(c)
    h_ref[...] = h
    fc_ref[...] = _fc(h, c, g_ref, uf_ref, bf_ref)


def _tc_leaf(x, g, w_iou, u_f, b_iou, b_f, off, n):
    ob = off // BLK
    shp = jax.ShapeDtypeStruct((n, E), F32)
    row = pl.BlockSpec((BLK, E), lambda i: (i, 0))
    return pl.pallas_call(
        _leaf_body,
        grid=(n // BLK,),
        in_specs=[
            pl.BlockSpec((BLK, E), lambda i: (i + ob, 0)),
            row,
            pl.BlockSpec((E, 3 * E), lambda i: (0, 0)),
            pl.BlockSpec((E, E), lambda i: (0, 0)),
            pl.BlockSpec((1, 3 * E), lambda i: (0, 0)),
            pl.BlockSpec((1, E), lambda i: (0, 0)),
        ],
        out_specs=[row, row],
        out_shape=[shp, shp],
    )(x, g, w_iou, u_f, b_iou, b_f)


def _cell_body(x_ref, hs_ref, fs_ref, g_ref, wiou_ref, uiou_ref, uf_ref,
               biou_ref, bf_ref, hfc_ref):
    iou = jnp.dot(x_ref[...], wiou_ref[...], preferred_element_type=F32)
    iou = iou + jnp.dot(hs_ref[...], uiou_ref[...], preferred_element_type=F32)
    iou = iou + biou_ref[...]
    i, o, u = jnp.split(iou, 3, axis=-1)
    c = jax.nn.sigmoid(i) * jnp.tanh(u) + fs_ref[...]
    h = jax.nn.sigmoid(o) * jnp.tanh(c)
    hfc_ref[...] = jnp.concatenate(
        [h, _fc(h, c, g_ref, uf_ref, bf_ref)], axis=-1)


def _tc_cell(x, hs, fs, g, w_iou, u_iou, u_f, b_iou, b_f, off, n, fused_in):
    ob = off // BLK
    col = 1 if fused_in else 0
    return pl.pallas_call(
        _cell_body,
        grid=(n // BLK,),
        in_specs=[
            pl.BlockSpec((BLK, E), lambda i: (i + ob, 0)),
            pl.BlockSpec((BLK, E), lambda i: (i, 0)),
            pl.BlockSpec((BLK, E), lambda i: (i, col)),
            pl.BlockSpec((BLK, E), lambda i: (i, 0)),
            pl.BlockSpec((E, 3 * E), lambda i: (0, 0)),
            pl.BlockSpec((E, 3 * E), lambda i: (0, 0)),
            pl.BlockSpec((E, E), lambda i: (0, 0)),
            pl.BlockSpec((1, 3 * E), lambda i: (0, 0)),
            pl.BlockSpec((1, E), lambda i: (0, 0)),
        ],
        out_specs=pl.BlockSpec((BLK, 2 * E), lambda i: (i, 0)),
        out_shape=jax.ShapeDtypeStruct((n, 2 * E), F32),
    )(x, hs, fs, g, w_iou, u_iou, u_f, b_iou, b_f)


def _root_body(x_ref, hs_ref, fs_ref, wiou_ref, uiou_ref, biou_ref, h_ref):
    iou = jnp.dot(x_ref[...], wiou_ref[...], preferred_element_type=F32)
    iou = iou + jnp.dot(hs_ref[...], uiou_ref[...], preferred_element_type=F32)
    iou = iou + biou_ref[...]
    i, o, u = jnp.split(iou, 3, axis=-1)
    c = jax.nn.sigmoid(i) * jnp.tanh(u) + fs_ref[...]
    h_ref[...] = jax.nn.sigmoid(o) * jnp.tanh(c)


def _tc_root(x, hsfs, w_iou, u_iou, b_iou, n):
    row = pl.BlockSpec((BLK, E), lambda i: (i, 0))
    return pl.pallas_call(
        _root_body,
        grid=(n // BLK,),
        in_specs=[
            row,
            pl.BlockSpec((BLK, E), lambda i: (i, 0)),
            pl.BlockSpec((BLK, E), lambda i: (i, 1)),
            pl.BlockSpec((E, 3 * E), lambda i: (0, 0)),
            pl.BlockSpec((E, 3 * E), lambda i: (0, 0)),
            pl.BlockSpec((1, 3 * E), lambda i: (0, 0)),
        ],
        out_specs=row,
        out_shape=jax.ShapeDtypeStruct((n, E), F32),
    )(x, hsfs, hsfs, w_iou, u_iou, b_iou)


def _pad_rows(x, p, fill):
    n = x.shape[0]
    return jnp.concatenate(
        [x.astype(jnp.int32), jnp.full((p - n,), fill, jnp.int32)])


@jax.jit
def kernel(tok0, tok1, tok2, tok3, parent1, parent2, parent3, embed_table,
           W_iou, U_iou, b_iou, W_f, U_f, b_f):
    toks = jnp.concatenate([
        _pad_rows(tok0, _P[0], 0), _pad_rows(tok1, _P[1], 0),
        _pad_rows(tok2, _P[2], 0), _pad_rows(tok3, _P[3], 0)])
    par1 = _pad_rows(parent1, _P[1], _L[0])
    par2 = _pad_rows(parent2, _P[2], _L[1])
    par3 = _pad_rows(parent3, _P[3], _L[2])

    b_iou2 = b_iou.reshape(1, 3 * E)
    b_f2 = b_f.reshape(1, E)

    x_all = _sc_gather(embed_table, toks, _T)
    p_f = _tc_pf(x_all, W_f)

    g3 = _sc_gather(p_f, par3 + _O[2], _P[3])
    g2 = _sc_gather(p_f, par2 + _O[1], _P[2])
    g1 = _sc_gather(p_f, par1 + _O[0], _P[1])

    h3, fc3 = _tc_leaf(x_all, g3, W_iou, U_f, b_iou2, b_f2, _O[3], _P[3])

    idx2, prm = _scatter_plan(par3, _P[2], _P[3])
    hs2 = _sc_scatter_add(h3, idx2, prm, _P[2])
    fs2 = _sc_scatter_add(fc3, idx2, prm, _P[2])
    hfc2 = _tc_cell(x_all, hs2, fs2, g2, W_iou, U_iou, U_f, b_iou2, b_f2,
                    _O[2], _P[2], fused_in=False)

    def ileave(par):
        return jnp.stack([2 * par, 2 * par + 1], axis=1).reshape(-1)

    idx2, prm = _scatter_plan(ileave(par2), 2 * _P[1], 2 * _P[2])
    hsfs1 = _sc_scatter_add(hfc2.reshape(-1, E), idx2, prm, 2 * _P[1])
    hfc1 = _tc_cell(x_all, hsfs1.reshape(_P[1], 2 * E),
                    hsfs1.reshape(_P[1], 2 * E), g1, W_iou, U_iou, U_f,
                    b_iou2, b_f2, _O[1], _P[1], fused_in=True)

    idx2, prm = _scatter_plan(ileave(par1), 2 * _P[0], 2 * _P[1])
    hsfs0 = _sc_scatter_add(hfc1.reshape(-1, E), idx2, prm, 2 * _P[0])
    h0 = _tc_root(x_all, hsfs0.reshape(_P[0], 2 * E), W_iou, U_iou, b_iou2,
                  _P[0])

    return jnp.concatenate(
        [h0[:_L[0]], hfc1[:_L[1], :E], hfc2[:_L[2], :E], h3[:_L[3]]], axis=0)


# speedup vs baseline: 1.0893x; 1.0893x over previous
import functools

import jax
import jax.numpy as jnp
from jax import lax
from jax.experimental import pallas as pl
from jax.experimental.pallas import tpu as pltpu
from jax.experimental.pallas import tpu_sc as plsc

F32 = jnp.float32
E = 128
BLK = 256
NC, NS = 2, 16
NW = NC * NS
CHUNK = 128

_L = (500, 4500, 15000, 80000)
_P = (512, 4608, 15360, 81920)
_O = (0, 512, 5120, 20480)
_T = 102400


def _mesh():
    return plsc.VectorSubcoreMesh(core_axis_name="c", subcore_axis_name="s")


_NB = 4


def _sc_gather(src, idx, n_rows):
    q = n_rows // NW
    nfull, r = divmod(q, CHUNK)
    nq, rem = divmod(nfull, _NB)

    scratch = [pltpu.VMEM((q,), jnp.int32)]
    scratch += [pltpu.VMEM((CHUNK, E), F32)] * _NB
    if r:
        scratch += [pltpu.VMEM((r, E), F32)]
    scratch += [pltpu.SemaphoreType.DMA] * (2 * _NB + 1)

    @functools.partial(
        pl.kernel, mesh=_mesh(),
        out_type=jax.ShapeDtypeStruct((n_rows, E), F32),
        scratch_types=scratch,
    )
    def k(src_hbm, idx_hbm, out_hbm, *sc):
        ia = sc[0]
        v = sc[1:1 + _NB]
        vt = sc[1 + _NB] if r else None
        sems = sc[-(2 * _NB + 1):]
        g = sems[:_NB]
        w = sems[_NB:2 * _NB]
        st = sems[2 * _NB]
        base = (lax.axis_index("c") * NS + lax.axis_index("s")) * q
        pltpu.sync_copy(idx_hbm.at[pl.ds(base, q)], ia)

        def start(c, b):
            pltpu.async_copy(
                src_hbm.at[ia.at[pl.ds(c * CHUNK, CHUNK)]], v[b], g[b])

        def wback(c, b):
            pltpu.async_copy(
                v[b], out_hbm.at[pl.ds(base + c * CHUNK, CHUNK)], w[b])

        def wait_g(b):
            pltpu.make_async_copy(src_hbm.at[pl.ds(0, CHUNK)], v[b],
                                  g[b]).wait()

        def wait_w(b):
            pltpu.make_async_copy(v[b], out_hbm.at[pl.ds(base, CHUNK)],
                                  w[b]).wait()

        if nq:
            for b in range(_NB):
                start(b, b)

            @pl.loop(0, nq - 1)
            def _(j):
                c = j * _NB
                for b in range(_NB):
                    wait_g(b)
                    wback(c + b, b)
                for b in range(_NB):
                    wait_w(b)
                    start(c + _NB + b, b)

            for b in range(_NB):
                wait_g(b)
                wback((nq - 1) * _NB + b, b)
            for b in range(rem):
                wait_w(b)
                start(nq * _NB + b, b)
            for b in range(rem):
                wait_g(b)
                wback(nq * _NB + b, b)
            for b in range(rem, _NB):
                wait_w(b)
            for b in range(rem):
                wait_w(b)
        else:
            for b in range(rem):
                start(b, b)
            for b in range(rem):
                wait_g(b)
                wback(b, b)
            for b in range(rem):
                wait_w(b)
        if r:
            t = nfull * CHUNK
            pltpu.async_copy(
                src_hbm.at[ia.at[pl.ds(t, r)]], vt, st).wait()
            pltpu.sync_copy(vt, out_hbm.at[pl.ds(base + t, r)])

    return k(src, idx)


def _scatter_plan(par, n_par, n_child):
    h = n_par // 2
    s = jnp.searchsorted(par, h).astype(jnp.int32)
    nc0 = (s + CHUNK - 1) // CHUNK
    base1 = s // CHUNK
    ntot = n_child // CHUNK
    idx0 = jnp.where(par < h, par, h)
    idx1 = jnp.where(par >= h, par - h, h)
    idx2 = jnp.concatenate([idx0, idx1])
    params = jnp.stack([nc0, base1] + [jnp.int32(0)] * 14)
    return idx2, params


def _sc_scatter_add(vals, idx2, params, n_par):
    n_child = vals.shape[0]
    h = n_par // 2
    z = h // NS
    ntot = n_child // CHUNK
    zeros = jnp.zeros((z, E), F32)

    scratch = [pltpu.VMEM((16,), jnp.int32),
               pltpu.VMEM((CHUNK,), jnp.int32), pltpu.VMEM((CHUNK, E), F32),
               pltpu.VMEM((CHUNK,), jnp.int32), pltpu.VMEM((CHUNK, E), F32),
               pltpu.VMEM_SHARED((h + 8, E), F32),
               pltpu.SemaphoreType.DMA, pltpu.SemaphoreType.DMA]

    @functools.partial(
        pl.kernel, mesh=_mesh(),
        out_type=jax.ShapeDtypeStruct((n_par, E), F32),
        scratch_types=scratch,
    )
    def k(vals_hbm, idx_hbm, params_hbm, zeros_hbm, out_hbm, *sc):
        pv, i0, v0, i1, v1, shared, s0, s1 = sc
        cid = lax.axis_index("c")
        sid = lax.axis_index("s")
        pltpu.sync_copy(params_hbm, pv)
        pltpu.sync_copy(zeros_hbm, shared.at[pl.ds(sid * z, z)])
        plsc.subcore_barrier()

        pvec = pv[...]
        nc0 = pvec[0]
        base1 = pvec[1]
        base_c = jnp.where(cid == 0, 0, base1)
        nc_c = jnp.where(cid == 0, nc0, ntot - base1)
        m = jnp.maximum(nc_c - sid + NS - 1, 0) // NS
        ioff = cid * n_child

        def pair(j, carry):
            g0 = base_c + sid + (2 * j) * NS
            a = g0 * CHUNK
            pltpu.sync_copy(idx_hbm.at[pl.ds(ioff + a, CHUNK)], i0)
            h0 = pltpu.async_copy(vals_hbm.at[pl.ds(a, CHUNK)], v0, s0)
            second = (2 * j + 1) < m

            @pl.when(second)
            def _():
                b = a + NS * CHUNK
                pltpu.sync_copy(idx_hbm.at[pl.ds(ioff + b, CHUNK)], i1)
                h1 = pltpu.async_copy(vals_hbm.at[pl.ds(b, CHUNK)], v1, s1)
                h0.wait()
                pltpu.sync_copy(v0, shared.at[i0], add=True)
                h1.wait()
                pltpu.sync_copy(v1, shared.at[i1], add=True)

            @pl.when(jnp.logical_not(second))
            def _():
                h0.wait()
                pltpu.sync_copy(v0, shared.at[i0], add=True)

            return carry

        lax.fori_loop(0, (m + 1) // 2, pair, 0)

        plsc.subcore_barrier()
        pltpu.sync_copy(shared.at[pl.ds(sid * z, z)],
                        out_hbm.at[pl.ds(cid * h + sid * z, z)])

    return k(vals, idx2, params, zeros)


_NPAR = _O[3]


def _pf_body(x_ref, wf_ref, pf_ref):
    pf_ref[...] = jnp.dot(x_ref[...], wf_ref[...], preferred_element_type=F32)


def _tc_pf(x, w_f):
    return pl.pallas_call(
        _pf_body,
        grid=(_NPAR // BLK,),
        in_specs=[
            pl.BlockSpec((BLK, E), lambda i: (i, 0)),
            pl.BlockSpec((E, E), lambda i: (0, 0)),
        ],
        out_specs=pl.BlockSpec((BLK, E), lambda i: (i, 0)),
        out_shape=jax.ShapeDtypeStruct((_NPAR, E), F32),
    )(x, w_f)


def _fc(h, c, g_ref, uf_ref, bf_ref):
    hu = jnp.dot(h, uf_ref[...], preferred_element_type=F32)
    return jax.nn.sigmoid(g_ref[...] + hu + bf_ref[...]) * c


def _leaf_body(x_ref, g_ref, wiou_ref, uf_ref, biou_ref, bf_ref,
               h_ref, fc_ref):
    iou = jnp.dot(x_ref[...], wiou_ref[...], preferred_element_type=F32)
    iou = iou + biou_ref[...]
    i, o, u = jnp.split(iou, 3, axis=-1)
    c = jax.nn.sigmoid(i) * jnp.tanh(u)
    h = jax.nn.sigmoid(o) * jnp.tanh(c)
    h_ref[...] = h
    fc_ref[...] = _fc(h, c, g_ref, uf_ref, bf_ref)


def _tc_leaf(x, g, w_iou, u_f, b_iou, b_f, off, n):
    ob = off // BLK
    shp = jax.ShapeDtypeStruct((n, E), F32)
    row = pl.BlockSpec((BLK, E), lambda i: (i, 0))
    return pl.pallas_call(
        _leaf_body,
        grid=(n // BLK,),
        in_specs=[
            pl.BlockSpec((BLK, E), lambda i: (i + ob, 0)),
            row,
            pl.BlockSpec((E, 3 * E), lambda i: (0, 0)),
            pl.BlockSpec((E, E), lambda i: (0, 0)),
            pl.BlockSpec((1, 3 * E), lambda i: (0, 0)),
            pl.BlockSpec((1, E), lambda i: (0, 0)),
        ],
        out_specs=[row, row],
        out_shape=[shp, shp],
    )(x, g, w_iou, u_f, b_iou, b_f)


def _cell_body(x_ref, hs_ref, fs_ref, g_ref, wiou_ref, uiou_ref, uf_ref,
               biou_ref, bf_ref, h_ref, fc_ref):
    iou = jnp.dot(x_ref[...], wiou_ref[...], preferred_element_type=F32)
    iou = iou + jnp.dot(hs_ref[...], uiou_ref[...], preferred_element_type=F32)
    iou = iou + biou_ref[...]
    i, o, u = jnp.split(iou, 3, axis=-1)
    c = jax.nn.sigmoid(i) * jnp.tanh(u) + fs_ref[...]
    h = jax.nn.sigmoid(o) * jnp.tanh(c)
    h_ref[...] = h
    fc_ref[...] = _fc(h, c, g_ref, uf_ref, bf_ref)


def _tc_cell(x, hs, fs, g, w_iou, u_iou, u_f, b_iou, b_f, off, n):
    ob = off // BLK
    shp = jax.ShapeDtypeStruct((n, E), F32)
    row = pl.BlockSpec((BLK, E), lambda i: (i, 0))
    return pl.pallas_call(
        _cell_body,
        grid=(n // BLK,),
        in_specs=[
            pl.BlockSpec((BLK, E), lambda i: (i + ob, 0)),
            row, row, row,
            pl.BlockSpec((E, 3 * E), lambda i: (0, 0)),
            pl.BlockSpec((E, 3 * E), lambda i: (0, 0)),
            pl.BlockSpec((E, E), lambda i: (0, 0)),
            pl.BlockSpec((1, 3 * E), lambda i: (0, 0)),
            pl.BlockSpec((1, E), lambda i: (0, 0)),
        ],
        out_specs=[row, row],
        out_shape=[shp, shp],
    )(x, hs, fs, g, w_iou, u_iou, u_f, b_iou, b_f)


def _root_body(x_ref, hs_ref, fs_ref, wiou_ref, uiou_ref, biou_ref, h_ref):
    iou = jnp.dot(x_ref[...], wiou_ref[...], preferred_element_type=F32)
    iou = iou + jnp.dot(hs_ref[...], uiou_ref[...], preferred_element_type=F32)
    iou = iou + biou_ref[...]
    i, o, u = jnp.split(iou, 3, axis=-1)
    c = jax.nn.sigmoid(i) * jnp.tanh(u) + fs_ref[...]
    h_ref[...] = jax.nn.sigmoid(o) * jnp.tanh(c)


def _tc_root(x, hs, fs, w_iou, u_iou, b_iou, n):
    row = pl.BlockSpec((BLK, E), lambda i: (i, 0))
    return pl.pallas_call(
        _root_body,
        grid=(n // BLK,),
        in_specs=[
            row, row, row,
            pl.BlockSpec((E, 3 * E), lambda i: (0, 0)),
            pl.BlockSpec((E, 3 * E), lambda i: (0, 0)),
            pl.BlockSpec((1, 3 * E), lambda i: (0, 0)),
        ],
        out_specs=row,
        out_shape=jax.ShapeDtypeStruct((n, E), F32),
    )(x, hs, fs, w_iou, u_iou, b_iou)


def _pad_rows(x, p, fill):
    n = x.shape[0]
    return jnp.concatenate(
        [x.astype(jnp.int32), jnp.full((p - n,), fill, jnp.int32)])


@jax.jit
def kernel(tok0, tok1, tok2, tok3, parent1, parent2, parent3, embed_table,
           W_iou, U_iou, b_iou, W_f, U_f, b_f):
    toks = jnp.concatenate([
        _pad_rows(tok0, _P[0], 0), _pad_rows(tok1, _P[1], 0),
        _pad_rows(tok2, _P[2], 0), _pad_rows(tok3, _P[3], 0)])
    par1 = _pad_rows(parent1, _P[1], _L[0])
    par2 = _pad_rows(parent2, _P[2], _L[1])
    par3 = _pad_rows(parent3, _P[3], _L[2])

    b_iou2 = b_iou.reshape(1, 3 * E)
    b_f2 = b_f.reshape(1, E)

    x_all = _sc_gather(embed_table, toks, _T)
    p_f = _tc_pf(x_all, W_f)

    g3 = _sc_gather(p_f, par3 + _O[2], _P[3])
    g2 = _sc_gather(p_f, par2 + _O[1], _P[2])
    g1 = _sc_gather(p_f, par1 + _O[0], _P[1])

    h3, fc3 = _tc_leaf(x_all, g3, W_iou, U_f, b_iou2, b_f2, _O[3], _P[3])

    def level_up(h_k, fc_k, par, g, lvl):
        n_p, off_p = _P[lvl], _O[lvl]
        idx2, prm = _scatter_plan(par, n_p, _P[lvl + 1])
        hs = _sc_scatter_add(h_k, idx2, prm, n_p)
        fs = _sc_scatter_add(fc_k, idx2, prm, n_p)
        if lvl == 0:
            return _tc_root(x_all, hs, fs, W_iou, U_iou, b_iou2, n_p), None
        return _tc_cell(x_all, hs, fs, g, W_iou, U_iou, U_f, b_iou2, b_f2,
                        off_p, n_p)

    h2, fc2 = level_up(h3, fc3, par3, g2, 2)
    h1, fc1 = level_up(h2, fc2, par2, g1, 1)
    h0, _ = level_up(h1, fc1, par1, None, 0)

    return jnp.concatenate(
        [h0[:_L[0]], h1[:_L[1]], h2[:_L[2]], h3[:_L[3]]], axis=0)


# speedup vs baseline: 1.0901x; 1.0007x over previous
import functools

import jax
import jax.numpy as jnp
from jax import lax
from jax.experimental import pallas as pl
from jax.experimental.pallas import tpu as pltpu
from jax.experimental.pallas import tpu_sc as plsc

F32 = jnp.float32
E = 128
BLK = 256
NC, NS = 2, 16
NW = NC * NS
CHUNK = 128

_L = (500, 4500, 15000, 80000)
_P = (512, 4608, 15360, 81920)
_O = (0, 512, 5120, 20480)
_T = 102400


def _mesh():
    return plsc.VectorSubcoreMesh(core_axis_name="c", subcore_axis_name="s")


_NB = 6


def _sc_gather(src, idx, n_rows):
    q = n_rows // NW
    nfull, r = divmod(q, CHUNK)
    nq, rem = divmod(nfull, _NB)

    scratch = [pltpu.VMEM((q,), jnp.int32)]
    scratch += [pltpu.VMEM((CHUNK, E), F32)] * _NB
    if r:
        scratch += [pltpu.VMEM((r, E), F32)]
    scratch += [pltpu.SemaphoreType.DMA] * (2 * _NB + 1)

    @functools.partial(
        pl.kernel, mesh=_mesh(),
        out_type=jax.ShapeDtypeStruct((n_rows, E), F32),
        scratch_types=scratch,
    )
    def k(src_hbm, idx_hbm, out_hbm, *sc):
        ia = sc[0]
        v = sc[1:1 + _NB]
        vt = sc[1 + _NB] if r else None
        sems = sc[-(2 * _NB + 1):]
        g = sems[:_NB]
        w = sems[_NB:2 * _NB]
        st = sems[2 * _NB]
        base = (lax.axis_index("c") * NS + lax.axis_index("s")) * q
        pltpu.sync_copy(idx_hbm.at[pl.ds(base, q)], ia)

        def start(c, b):
            pltpu.async_copy(
                src_hbm.at[ia.at[pl.ds(c * CHUNK, CHUNK)]], v[b], g[b])

        def wback(c, b):
            pltpu.async_copy(
                v[b], out_hbm.at[pl.ds(base + c * CHUNK, CHUNK)], w[b])

        def wait_g(b):
            pltpu.make_async_copy(src_hbm.at[pl.ds(0, CHUNK)], v[b],
                                  g[b]).wait()

        def wait_w(b):
            pltpu.make_async_copy(v[b], out_hbm.at[pl.ds(base, CHUNK)],
                                  w[b]).wait()

        if nq:
            for b in range(_NB):
                start(b, b)

            @pl.loop(0, nq - 1)
            def _(j):
                c = j * _NB
                for b in range(_NB):
                    wait_g(b)
                    wback(c + b, b)
                for b in range(_NB):
                    wait_w(b)
                    start(c + _NB + b, b)

            for b in range(_NB):
                wait_g(b)
                wback((nq - 1) * _NB + b, b)
            for b in range(rem):
                wait_w(b)
                start(nq * _NB + b, b)
            for b in range(rem):
                wait_g(b)
                wback(nq * _NB + b, b)
            for b in range(rem, _NB):
                wait_w(b)
            for b in range(rem):
                wait_w(b)
        else:
            for b in range(rem):
                start(b, b)
            for b in range(rem):
                wait_g(b)
                wback(b, b)
            for b in range(rem):
                wait_w(b)
        if r:
            t = nfull * CHUNK
            pltpu.async_copy(
                src_hbm.at[ia.at[pl.ds(t, r)]], vt, st).wait()
            pltpu.sync_copy(vt, out_hbm.at[pl.ds(base + t, r)])

    return k(src, idx)


def _scatter_plan(par, n_par, n_child):
    h = n_par // 2
    s = jnp.searchsorted(par, h).astype(jnp.int32)
    nc0 = (s + CHUNK - 1) // CHUNK
    base1 = s // CHUNK
    ntot = n_child // CHUNK
    idx0 = jnp.where(par < h, par, h)
    idx1 = jnp.where(par >= h, par - h, h)
    idx2 = jnp.concatenate([idx0, idx1])
    params = jnp.stack([nc0, base1] + [jnp.int32(0)] * 14)
    return idx2, params


def _sc_scatter_add(vals, idx2, params, n_par):
    n_child = vals.shape[0]
    h = n_par // 2
    z = h // NS
    ntot = n_child // CHUNK
    zeros = jnp.zeros((z, E), F32)

    scratch = [pltpu.VMEM((16,), jnp.int32),
               pltpu.VMEM((CHUNK,), jnp.int32), pltpu.VMEM((CHUNK, E), F32),
               pltpu.VMEM((CHUNK,), jnp.int32), pltpu.VMEM((CHUNK, E), F32),
               pltpu.VMEM_SHARED((h + 8, E), F32),
               pltpu.SemaphoreType.DMA, pltpu.SemaphoreType.DMA]

    @functools.partial(
        pl.kernel, mesh=_mesh(),
        out_type=jax.ShapeDtypeStruct((n_par, E), F32),
        scratch_types=scratch,
    )
    def k(vals_hbm, idx_hbm, params_hbm, zeros_hbm, out_hbm, *sc):
        pv, i0, v0, i1, v1, shared, s0, s1 = sc
        cid = lax.axis_index("c")
        sid = lax.axis_index("s")
        pltpu.sync_copy(params_hbm, pv)
        pltpu.sync_copy(zeros_hbm, shared.at[pl.ds(sid * z, z)])
        plsc.subcore_barrier()

        pvec = pv[...]
        nc0 = pvec[0]
        base1 = pvec[1]
        base_c = jnp.where(cid == 0, 0, base1)
        nc_c = jnp.where(cid == 0, nc0, ntot - base1)
        m = jnp.maximum(nc_c - sid + NS - 1, 0) // NS
        ioff = cid * n_child

        def pair(j, carry):
            g0 = base_c + sid + (2 * j) * NS
            a = g0 * CHUNK
            pltpu.sync_copy(idx_hbm.at[pl.ds(ioff + a, CHUNK)], i0)
            h0 = pltpu.async_copy(vals_hbm.at[pl.ds(a, CHUNK)], v0, s0)
            second = (2 * j + 1) < m

            @pl.when(second)
            def _():
                b = a + NS * CHUNK
                pltpu.sync_copy(idx_hbm.at[pl.ds(ioff + b, CHUNK)], i1)
                h1 = pltpu.async_copy(vals_hbm.at[pl.ds(b, CHUNK)], v1, s1)
                h0.wait()
                pltpu.sync_copy(v0, shared.at[i0], add=True)
                h1.wait()
                pltpu.sync_copy(v1, shared.at[i1], add=True)

            @pl.when(jnp.logical_not(second))
            def _():
                h0.wait()
                pltpu.sync_copy(v0, shared.at[i0], add=True)

            return carry

        lax.fori_loop(0, (m + 1) // 2, pair, 0)

        plsc.subcore_barrier()
        pltpu.sync_copy(shared.at[pl.ds(sid * z, z)],
                        out_hbm.at[pl.ds(cid * h + sid * z, z)])

    return k(vals, idx2, params, zeros)


_NPAR = _O[3]


def _pf_body(x_ref, wf_ref, pf_ref):
    pf_ref[...] = jnp.dot(x_ref[...], wf_ref[...], preferred_element_type=F32)


def _tc_pf(x, w_f):
    return pl.pallas_call(
        _pf_body,
        grid=(_NPAR // BLK,),
        in_specs=[
            pl.BlockSpec((BLK, E), lambda i: (i, 0)),
            pl.BlockSpec((E, E), lambda i: (0, 0)),
        ],
        out_specs=pl.BlockSpec((BLK, E), lambda i: (i, 0)),
        out_shape=jax.ShapeDtypeStruct((_NPAR, E), F32),
    )(x, w_f)


def _fc(h, c, g_ref, uf_ref, bf_ref):
    hu = jnp.dot(h, uf_ref[...], preferred_element_type=F32)
    return jax.nn.sigmoid(g_ref[...] + hu + bf_ref[...]) * c


def _leaf_body(x_ref, g_ref, wiou_ref, uf_ref, biou_ref, bf_ref,
               h_ref, fc_ref):
    iou = jnp.dot(x_ref[...], wiou_ref[...], preferred_element_type=F32)
    iou = iou + biou_ref[...]
    i, o, u = jnp.split(iou, 3, axis=-1)
    c = jax.nn.sigmoid(i) * jnp.tanh(u)
    h = jax.nn.sigmoid(o) * jnp.tanh(c)
    h_ref[...] = h
    fc_ref[...] = _fc(h, c, g_ref, uf_ref, bf_ref)


def _tc_leaf(x, g, w_iou, u_f, b_iou, b_f, off, n):
    ob = off // BLK
    shp = jax.ShapeDtypeStruct((n, E), F32)
    row = pl.BlockSpec((BLK, E), lambda i: (i, 0))
    return pl.pallas_call(
        _leaf_body,
        grid=(n // BLK,),
        in_specs=[
            pl.BlockSpec((BLK, E), lambda i: (i + ob, 0)),
            row,
            pl.BlockSpec((E, 3 * E), lambda i: (0, 0)),
            pl.BlockSpec((E, E), lambda i: (0, 0)),
            pl.BlockSpec((1, 3 * E), lambda i: (0, 0)),
            pl.BlockSpec((1, E), lambda i: (0, 0)),
        ],
        out_specs=[row, row],
        out_shape=[shp, shp],
    )(x, g, w_iou, u_f, b_iou, b_f)


def _cell_body(x_ref, hs_ref, fs_ref, g_ref, wiou_ref, uiou_ref, uf_ref,
               biou_ref, bf_ref, h_ref, fc_ref):
    iou = jnp.dot(x_ref[...], wiou_ref[...], preferred_element_type=F32)
    iou = iou + jnp.dot(hs_ref[...], uiou_ref[...], preferred_element_type=F32)
    iou = iou + biou_ref[...]
    i, o, u = jnp.split(iou, 3, axis=-1)
    c = jax.nn.sigmoid(i) * jnp.tanh(u) + fs_ref[...]
    h = jax.nn.sigmoid(o) * jnp.tanh(c)
    h_ref[...] = h
    fc_ref[...] = _fc(h, c, g_ref, uf_ref, bf_ref)


def _tc_cell(x, hs, fs, g, w_iou, u_iou, u_f, b_iou, b_f, off, n):
    ob = off // BLK
    shp = jax.ShapeDtypeStruct((n, E), F32)
    row = pl.BlockSpec((BLK, E), lambda i: (i, 0))
    return pl.pallas_call(
        _cell_body,
        grid=(n // BLK,),
        in_specs=[
            pl.BlockSpec((BLK, E), lambda i: (i + ob, 0)),
            row, row, row,
            pl.BlockSpec((E, 3 * E), lambda i: (0, 0)),
            pl.BlockSpec((E, 3 * E), lambda i: (0, 0)),
            pl.BlockSpec((E, E), lambda i: (0, 0)),
            pl.BlockSpec((1, 3 * E), lambda i: (0, 0)),
            pl.BlockSpec((1, E), lambda i: (0, 0)),
        ],
        out_specs=[row, row],
        out_shape=[shp, shp],
    )(x, hs, fs, g, w_iou, u_iou, u_f, b_iou, b_f)


def _root_body(x_ref, hs_ref, fs_ref, wiou_ref, uiou_ref, biou_ref, h_ref):
    iou = jnp.dot(x_ref[...], wiou_ref[...], preferred_element_type=F32)
    iou = iou + jnp.dot(hs_ref[...], uiou_ref[...], preferred_element_type=F32)
    iou = iou + biou_ref[...]
    i, o, u = jnp.split(iou, 3, axis=-1)
    c = jax.nn.sigmoid(i) * jnp.tanh(u) + fs_ref[...]
    h_ref[...] = jax.nn.sigmoid(o) * jnp.tanh(c)


def _tc_root(x, hs, fs, w_iou, u_iou, b_iou, n):
    row = pl.BlockSpec((BLK, E), lambda i: (i, 0))
    return pl.pallas_call(
        _root_body,
        grid=(n // BLK,),
        in_specs=[
            row, row, row,
            pl.BlockSpec((E, 3 * E), lambda i: (0, 0)),
            pl.BlockSpec((E, 3 * E), lambda i: (0, 0)),
            pl.BlockSpec((1, 3 * E), lambda i: (0, 0)),
        ],
        out_specs=row,
        out_shape=jax.ShapeDtypeStruct((n, E), F32),
    )(x, hs, fs, w_iou, u_iou, b_iou)


def _pad_rows(x, p, fill):
    n = x.shape[0]
    return jnp.concatenate(
        [x.astype(jnp.int32), jnp.full((p - n,), fill, jnp.int32)])


@jax.jit
def kernel(tok0, tok1, tok2, tok3, parent1, parent2, parent3, embed_table,
           W_iou, U_iou, b_iou, W_f, U_f, b_f):
    toks = jnp.concatenate([
        _pad_rows(tok0, _P[0], 0), _pad_rows(tok1, _P[1], 0),
        _pad_rows(tok2, _P[2], 0), _pad_rows(tok3, _P[3], 0)])
    par1 = _pad_rows(parent1, _P[1], _L[0])
    par2 = _pad_rows(parent2, _P[2], _L[1])
    par3 = _pad_rows(parent3, _P[3], _L[2])

    b_iou2 = b_iou.reshape(1, 3 * E)
    b_f2 = b_f.reshape(1, E)

    x_all = _sc_gather(embed_table, toks, _T)
    p_f = _tc_pf(x_all, W_f)

    g3 = _sc_gather(p_f, par3 + _O[2], _P[3])
    g2 = _sc_gather(p_f, par2 + _O[1], _P[2])
    g1 = _sc_gather(p_f, par1 + _O[0], _P[1])

    h3, fc3 = _tc_leaf(x_all, g3, W_iou, U_f, b_iou2, b_f2, _O[3], _P[3])

    def level_up(h_k, fc_k, par, g, lvl):
        n_p, off_p = _P[lvl], _O[lvl]
        idx2, prm = _scatter_plan(par, n_p, _P[lvl + 1])
        hs = _sc_scatter_add(h_k, idx2, prm, n_p)
        fs = _sc_scatter_add(fc_k, idx2, prm, n_p)
        if lvl == 0:
            return _tc_root(x_all, hs, fs, W_iou, U_iou, b_iou2, n_p), None
        return _tc_cell(x_all, hs, fs, g, W_iou, U_iou, U_f, b_iou2, b_f2,
                        off_p, n_p)

    h2, fc2 = level_up(h3, fc3, par3, g2, 2)
    h1, fc1 = level_up(h2, fc2, par2, g1, 1)
    h0, _ = level_up(h1, fc1, par1, None, 0)

    return jnp.concatenate(
        [h0[:_L[0]], h1[:_L[1]], h2[:_L[2]], h3[:_L[3]]], axis=0)
